# Initial kernel scaffold; baseline (speedup 1.0000x reference)
import jax
import jax.numpy as jnp
from jax.experimental import pallas as pl


def kernel(x, edges_index, W1, att_src1, att_dst1, b1, W2, att_src2, att_dst2, b2):
    raise NotImplementedError("write your pallas kernel here")


# speedup vs baseline: 28.6347x; 28.6347x over previous
import functools

import jax
import jax.numpy as jnp
from jax import lax
from jax.experimental import pallas as pl
from jax.experimental.pallas import tpu as pltpu
from jax.experimental.pallas import tpu_sc as plsc

_NC = 2
_NT = 16
_L = 16
_CH = 64
_BN = 2048


def _dense1_body(x_ref, w_ref, as_ref, ad_ref, h_ref, a1_ref, a2_ref, m_ref):
    i = pl.program_id(0)
    h = jnp.dot(x_ref[...], w_ref[...], preferred_element_type=jnp.float32)
    h_ref[...] = h
    s = jnp.sum(h * as_ref[...], axis=1)
    t = jnp.sum(h * ad_ref[...], axis=1)
    a1_ref[...] = s
    a2_ref[...] = t

    @pl.when(i == 0)
    def _init():
        m_ref[...] = jnp.full((2, 128), -1e30, jnp.float32)

    sm = jnp.max(s.reshape(-1, 128), axis=0, keepdims=True)
    tm = jnp.max(t.reshape(-1, 128), axis=0, keepdims=True)
    m_ref[...] = jnp.maximum(m_ref[...], jnp.concatenate([sm, tm], axis=0))


def _dense1(xp, w1, as_row, ad_row):
    npad, dfeat = xp.shape
    d = w1.shape[1]
    return pl.pallas_call(
        _dense1_body,
        grid=(npad // _BN,),
        in_specs=[
            pl.BlockSpec((_BN, dfeat), lambda i: (i, 0)),
            pl.BlockSpec((dfeat, d), lambda i: (0, 0)),
            pl.BlockSpec((1, d), lambda i: (0, 0)),
            pl.BlockSpec((1, d), lambda i: (0, 0)),
        ],
        out_specs=[
            pl.BlockSpec((_BN, d), lambda i: (i, 0)),
            pl.BlockSpec((_BN,), lambda i: (i,)),
            pl.BlockSpec((_BN,), lambda i: (i,)),
            pl.BlockSpec((2, 128), lambda i: (0, 0)),
        ],
        out_shape=[
            jax.ShapeDtypeStruct((npad, d), jnp.float32),
            jax.ShapeDtypeStruct((npad,), jnp.float32),
            jax.ShapeDtypeStruct((npad,), jnp.float32),
            jax.ShapeDtypeStruct((2, 128), jnp.float32),
        ],
    )(xp, w1, as_row, ad_row)


def _dense2_body(p0_ref, p1_ref, b_ref, w_ref, as_ref, ad_ref,
                 h_ref, a1_ref, a2_ref, m_ref):
    i = pl.program_id(0)
    act = jnp.maximum(p0_ref[...] + p1_ref[...] + b_ref[...], 0.0)
    h = jnp.dot(act, w_ref[...], preferred_element_type=jnp.float32)
    h_ref[...] = h
    s = jnp.sum(h * as_ref[...], axis=1)
    t = jnp.sum(h * ad_ref[...], axis=1)
    a1_ref[...] = s
    a2_ref[...] = t

    @pl.when(i == 0)
    def _init():
        m_ref[...] = jnp.full((2, 128), -1e30, jnp.float32)

    sm = jnp.max(s.reshape(-1, 128), axis=0, keepdims=True)
    tm = jnp.max(t.reshape(-1, 128), axis=0, keepdims=True)
    m_ref[...] = jnp.maximum(m_ref[...], jnp.concatenate([sm, tm], axis=0))


def _dense2(p0, p1, b_row, w2, as_row, ad_row):
    npad, dhid = p0.shape
    d = w2.shape[1]
    return pl.pallas_call(
        _dense2_body,
        grid=(npad // _BN,),
        in_specs=[
            pl.BlockSpec((_BN, dhid), lambda i: (i, 0)),
            pl.BlockSpec((_BN, dhid), lambda i: (i, 0)),
            pl.BlockSpec((1, dhid), lambda i: (0, 0)),
            pl.BlockSpec((dhid, d), lambda i: (0, 0)),
            pl.BlockSpec((1, d), lambda i: (0, 0)),
            pl.BlockSpec((1, d), lambda i: (0, 0)),
        ],
        out_specs=[
            pl.BlockSpec((_BN, d), lambda i: (i, 0)),
            pl.BlockSpec((_BN,), lambda i: (i,)),
            pl.BlockSpec((_BN,), lambda i: (i,)),
            pl.BlockSpec((2, 128), lambda i: (0, 0)),
        ],
        out_shape=[
            jax.ShapeDtypeStruct((npad, d), jnp.float32),
            jax.ShapeDtypeStruct((npad,), jnp.float32),
            jax.ShapeDtypeStruct((npad,), jnp.float32),
            jax.ShapeDtypeStruct((2, 128), jnp.float32),
        ],
    )(p0, p1, b_row, w2, as_row, ad_row)


def _combine_body(q0_ref, q1_ref, b_ref, o_ref):
    o_ref[...] = q0_ref[...] + q1_ref[...] + b_ref[...]


def _combine(q0, q1, b_row):
    npad, d = q0.shape
    return pl.pallas_call(
        _combine_body,
        grid=(npad // _BN,),
        in_specs=[
            pl.BlockSpec((_BN, d), lambda i: (i, 0)),
            pl.BlockSpec((_BN, d), lambda i: (i, 0)),
            pl.BlockSpec((1, d), lambda i: (0, 0)),
        ],
        out_specs=pl.BlockSpec((_BN, d), lambda i: (i, 0)),
        out_shape=jax.ShapeDtypeStruct((npad, d), jnp.float32),
    )(q0, q1, b_row)


_RB = 27


@functools.cache
def _make_edge_kernel(npad, d, echunks, e_total):
    nrows = echunks // _NT
    half = nrows // 2
    out_rows = npad // _NT
    nq = out_rows // _CH
    ng = _CH // _L
    assert nrows % _RB == 0 and half % _RB == 0
    mesh = plsc.VectorSubcoreMesh(
        core_axis_name="c", subcore_axis_name="s",
        num_cores=_NC, num_subcores=_NT)

    scratch = [
        pltpu.VMEM((npad,), jnp.float32),
        pltpu.VMEM((npad,), jnp.float32),
        pltpu.VMEM((npad,), jnp.float32),
        pltpu.VMEM((_RB, _CH), jnp.int32),
        pltpu.VMEM((_RB, _CH), jnp.int32),
        pltpu.VMEM((_RB, _CH), jnp.float32),
        pltpu.VMEM((_CH, d), jnp.float32),
        pltpu.VMEM((_CH,), jnp.float32),
        pltpu.VMEM((_L,), jnp.float32),
        pltpu.VMEM((out_rows,), jnp.float32),
        pltpu.VMEM_SHARED((npad, d), jnp.float32),
        pltpu.VMEM_SHARED((npad,), jnp.float32),
        pltpu.SemaphoreType.DMA,
    ]

    @functools.partial(
        pl.kernel,
        out_type=jax.ShapeDtypeStruct((_NC, npad, d), jnp.float32),
        mesh=mesh,
        scratch_types=scratch,
        compiler_params=pltpu.CompilerParams(
            needs_layout_passes=False, use_tc_tiling_on_sc=False),
    )
    def edge_kernel(h_hbm, asrc_hbm, adst_hbm, src_hbm, dst_hbm,
                    m_hbm, out_hbm, asrc_v, adst_v, denom_v, srcb_v,
                    dstb_v, p_blk, rows_v, w_v, m_v, z_v, out_sh, den_sh,
                    sem):
        c = lax.axis_index("c")
        s = lax.axis_index("s")

        pltpu.sync_copy(asrc_hbm, asrc_v)
        pltpu.sync_copy(adst_hbm, adst_v)
        pltpu.sync_copy(m_hbm, m_v)

        def _zz(i, carry):
            z_v[pl.ds(i * _L, _L)] = jnp.zeros((_L,), jnp.float32)
            return carry
        lax.fori_loop(0, out_rows // _L, _zz, 0)
        pltpu.sync_copy(z_v, den_sh.at[pl.ds(s * out_rows, out_rows)])

        def _zr(r, carry):
            def _zc(g, carry2):
                rows_v[r, pl.ds(g * _L, _L)] = jnp.zeros((_L,), jnp.float32)
                return carry2
            return lax.fori_loop(0, d // _L, _zc, carry)
        lax.fori_loop(0, _CH, _zr, 0)
        for q in range(nq):
            pltpu.sync_copy(
                rows_v, out_sh.at[pl.ds(s * out_rows + q * _CH, _CH)])

        plsc.subcore_barrier()

        mvec = m_v[...]

        def _edge_p(rr, g, eid0):
            srcv = srcb_v[rr, pl.ds(g * _L, _L)]
            dstv = dstb_v[rr, pl.ds(g * _L, _L)]
            av = (plsc.load_gather(asrc_v, [srcv])
                  + plsc.load_gather(adst_v, [dstv]))
            av = jnp.maximum(av, 0.2 * av)
            p = jnp.exp(av - mvec)
            eid = eid0 + lax.iota(jnp.int32, _L)
            return jnp.where(eid < e_total, p, 0.0), dstv

        def _body_a(b, carry):
            row0 = b * _RB
            pltpu.sync_copy(src_hbm.at[s, pl.ds(row0, _RB)], srcb_v)
            pltpu.sync_copy(dst_hbm.at[s, pl.ds(row0, _RB)], dstb_v)
            eid_base = (s * nrows + row0) * _CH

            def _row_a(rr, carry2):
                for g in range(ng):
                    p, _ = _edge_p(rr, g, eid_base + rr * _CH + g * _L)
                    p_blk[rr, pl.ds(g * _L, _L)] = p
                pltpu.sync_copy(p_blk.at[rr], den_sh.at[dstb_v.at[rr]],
                                add=True)
                return carry2
            lax.fori_loop(0, _RB, _row_a, 0)
            return carry
        lax.fori_loop(0, nrows // _RB, _body_a, 0)

        plsc.subcore_barrier()
        pltpu.sync_copy(den_sh, denom_v)

        def _body_b(b, carry):
            row0 = c * half + b * _RB
            pltpu.sync_copy(src_hbm.at[s, pl.ds(row0, _RB)], srcb_v)
            pltpu.sync_copy(dst_hbm.at[s, pl.ds(row0, _RB)], dstb_v)
            eid_base = (s * nrows + row0) * _CH

            def _chunk(rr, carry2):
                pltpu.async_copy(
                    h_hbm.at[srcb_v.at[rr]], rows_v, sem).wait()
                for g in range(ng):
                    p, dstv = _edge_p(rr, g, eid_base + rr * _CH + g * _L)
                    dn = plsc.load_gather(denom_v, [dstv])
                    w_v[pl.ds(g * _L, _L)] = p / dn

                def _scale(g16, carry3):
                    wg = w_v[pl.ds(g16 * _L, _L)]
                    for lane in range(_L):
                        wk = wg[lane]
                        k2 = g16 * _L + lane
                        for g2 in range(d // _L):
                            sl = pl.ds(g2 * _L, _L)
                            rows_v[k2, sl] = rows_v[k2, sl] * wk
                    return carry3
                lax.fori_loop(0, ng, _scale, 0)
                pltpu.sync_copy(rows_v, out_sh.at[dstb_v.at[rr]], add=True)
                return carry2
            lax.fori_loop(0, _RB, _chunk, 0)
            return carry
        lax.fori_loop(0, half // _RB, _body_b, 0)

        plsc.subcore_barrier()

        base = s * out_rows
        for q in range(nq):
            off = base + q * _CH
            pltpu.sync_copy(out_sh.at[pl.ds(off, _CH)], rows_v)
            pltpu.sync_copy(rows_v, out_hbm.at[c, pl.ds(off, _CH)])

    return edge_kernel


def _splat_bound(m):
    big = jnp.maximum(0.0, jnp.max(m[0]) + jnp.max(m[1]))
    return jnp.full((_L,), big, jnp.float32)


def kernel(x, edges_index, W1, att_src1, att_dst1, b1,
           W2, att_src2, att_dst2, b2):
    n, dfeat = x.shape
    dhid = W1.shape[1]
    ncls = W2.shape[1]
    d2p = ((ncls + _L - 1) // _L) * _L

    npad = ((n + _BN - 1) // _BN) * _BN
    assert npad % (_NT * _CH) == 0

    e0 = edges_index.shape[1]
    e_total = e0 + n
    grp = _NC * _NT * _CH
    epad = ((e_total + grp - 1) // grp) * grp
    echunks = epad // _CH

    pad_cnt = epad - e_total
    loop_idx = jnp.arange(n, dtype=jnp.int32)
    pad_idx = jnp.arange(pad_cnt, dtype=jnp.int32) % n
    src = jnp.concatenate([edges_index[0].astype(jnp.int32), loop_idx, pad_idx])
    dst = jnp.concatenate([edges_index[1].astype(jnp.int32), loop_idx, pad_idx])
    nrows = echunks // _NT
    src3 = src.reshape(_NT, nrows, _CH)
    dst3 = dst.reshape(_NT, nrows, _CH)

    xp = jnp.pad(x, ((0, npad - n), (0, 0)))

    h1, as1, ad1, m1 = _dense1(xp, W1,
                               att_src1.reshape(1, dhid),
                               att_dst1.reshape(1, dhid))
    part1 = _make_edge_kernel(npad, dhid, echunks, e_total)(
        h1, as1, ad1, src3, dst3, _splat_bound(m1))

    w2p = jnp.pad(W2, ((0, 0), (0, d2p - ncls)))
    as2p = jnp.pad(att_src2, (0, d2p - ncls)).reshape(1, d2p)
    ad2p = jnp.pad(att_dst2, (0, d2p - ncls)).reshape(1, d2p)
    h2, as2, ad2, m2 = _dense2(part1[0], part1[1], b1.reshape(1, dhid),
                               w2p, as2p, ad2p)
    part2 = _make_edge_kernel(npad, d2p, echunks, e_total)(
        h2, as2, ad2, src3, dst3, _splat_bound(m2))

    b2p = jnp.pad(b2, (0, d2p - ncls)).reshape(1, d2p)
    out = _combine(part2[0], part2[1], b2p)
    return out[:n, :ncls]


# speedup vs baseline: 35.4288x; 1.2373x over previous
import functools

import jax
import jax.numpy as jnp
from jax import lax
from jax.experimental import pallas as pl
from jax.experimental.pallas import tpu as pltpu
from jax.experimental.pallas import tpu_sc as plsc

_NC = 2
_NT = 16
_L = 16
_CH = 64
_CB = 32
_BN = 2048


def _dense1_body(x_ref, w_ref, as_ref, ad_ref, h_ref, a1_ref, a2_ref, m_ref):
    i = pl.program_id(0)
    h = jnp.dot(x_ref[...], w_ref[...], preferred_element_type=jnp.float32)
    h_ref[...] = h
    s = jnp.sum(h * as_ref[...], axis=1)
    t = jnp.sum(h * ad_ref[...], axis=1)
    a1_ref[...] = s
    a2_ref[...] = t

    @pl.when(i == 0)
    def _init():
        m_ref[...] = jnp.full((2, 128), -1e30, jnp.float32)

    sm = jnp.max(s.reshape(-1, 128), axis=0, keepdims=True)
    tm = jnp.max(t.reshape(-1, 128), axis=0, keepdims=True)
    m_ref[...] = jnp.maximum(m_ref[...], jnp.concatenate([sm, tm], axis=0))


def _dense1(xp, w1, as_row, ad_row):
    npad, dfeat = xp.shape
    d = w1.shape[1]
    return pl.pallas_call(
        _dense1_body,
        grid=(npad // _BN,),
        in_specs=[
            pl.BlockSpec((_BN, dfeat), lambda i: (i, 0)),
            pl.BlockSpec((dfeat, d), lambda i: (0, 0)),
            pl.BlockSpec((1, d), lambda i: (0, 0)),
            pl.BlockSpec((1, d), lambda i: (0, 0)),
        ],
        out_specs=[
            pl.BlockSpec((_BN, d), lambda i: (i, 0)),
            pl.BlockSpec((_BN,), lambda i: (i,)),
            pl.BlockSpec((_BN,), lambda i: (i,)),
            pl.BlockSpec((2, 128), lambda i: (0, 0)),
        ],
        out_shape=[
            jax.ShapeDtypeStruct((npad, d), jnp.float32),
            jax.ShapeDtypeStruct((npad,), jnp.float32),
            jax.ShapeDtypeStruct((npad,), jnp.float32),
            jax.ShapeDtypeStruct((2, 128), jnp.float32),
        ],
    )(xp, w1, as_row, ad_row)


def _dense2_body(p0_ref, p1_ref, b_ref, w_ref, as_ref, ad_ref,
                 h_ref, a1_ref, a2_ref, m_ref):
    i = pl.program_id(0)
    act = jnp.maximum(p0_ref[...] + p1_ref[...] + b_ref[...], 0.0)
    h = jnp.dot(act, w_ref[...], preferred_element_type=jnp.float32)
    h_ref[...] = h
    s = jnp.sum(h * as_ref[...], axis=1)
    t = jnp.sum(h * ad_ref[...], axis=1)
    a1_ref[...] = s
    a2_ref[...] = t

    @pl.when(i == 0)
    def _init():
        m_ref[...] = jnp.full((2, 128), -1e30, jnp.float32)

    sm = jnp.max(s.reshape(-1, 128), axis=0, keepdims=True)
    tm = jnp.max(t.reshape(-1, 128), axis=0, keepdims=True)
    m_ref[...] = jnp.maximum(m_ref[...], jnp.concatenate([sm, tm], axis=0))


def _dense2(p0, p1, b_row, w2, as_row, ad_row):
    npad, dhid = p0.shape
    d = w2.shape[1]
    return pl.pallas_call(
        _dense2_body,
        grid=(npad // _BN,),
        in_specs=[
            pl.BlockSpec((_BN, dhid), lambda i: (i, 0)),
            pl.BlockSpec((_BN, dhid), lambda i: (i, 0)),
            pl.BlockSpec((1, dhid), lambda i: (0, 0)),
            pl.BlockSpec((dhid, d), lambda i: (0, 0)),
            pl.BlockSpec((1, d), lambda i: (0, 0)),
            pl.BlockSpec((1, d), lambda i: (0, 0)),
        ],
        out_specs=[
            pl.BlockSpec((_BN, d), lambda i: (i, 0)),
            pl.BlockSpec((_BN,), lambda i: (i,)),
            pl.BlockSpec((_BN,), lambda i: (i,)),
            pl.BlockSpec((2, 128), lambda i: (0, 0)),
        ],
        out_shape=[
            jax.ShapeDtypeStruct((npad, d), jnp.float32),
            jax.ShapeDtypeStruct((npad,), jnp.float32),
            jax.ShapeDtypeStruct((npad,), jnp.float32),
            jax.ShapeDtypeStruct((2, 128), jnp.float32),
        ],
    )(p0, p1, b_row, w2, as_row, ad_row)


def _combine_body(q0_ref, q1_ref, b_ref, o_ref):
    o_ref[...] = q0_ref[...] + q1_ref[...] + b_ref[...]


def _combine(q0, q1, b_row):
    npad, d = q0.shape
    return pl.pallas_call(
        _combine_body,
        grid=(npad // _BN,),
        in_specs=[
            pl.BlockSpec((_BN, d), lambda i: (i, 0)),
            pl.BlockSpec((_BN, d), lambda i: (i, 0)),
            pl.BlockSpec((1, d), lambda i: (0, 0)),
        ],
        out_specs=pl.BlockSpec((_BN, d), lambda i: (i, 0)),
        out_shape=jax.ShapeDtypeStruct((npad, d), jnp.float32),
    )(q0, q1, b_row)


_RB = 27


@functools.cache
def _make_edge_kernel(npad, d, echunks, e_total):
    nrows = echunks // _NT
    half = nrows // 2
    out_rows = npad // _NT
    ng = _CH // _L
    assert nrows % _RB == 0 and half % _RB == 0
    mesh = plsc.VectorSubcoreMesh(
        core_axis_name="c", subcore_axis_name="s",
        num_cores=_NC, num_subcores=_NT)

    scratch = [
        pltpu.VMEM((npad,), jnp.float32),
        pltpu.VMEM((npad,), jnp.float32),
        pltpu.VMEM((npad,), jnp.float32),
        pltpu.VMEM((_RB, _CH), jnp.int32),
        pltpu.VMEM((_RB, _CH), jnp.int32),
        pltpu.VMEM((_RB, _CH), jnp.float32),
        pltpu.VMEM((2, _CB, d), jnp.float32),
        pltpu.VMEM((_CB,), jnp.float32),
        pltpu.VMEM((_L,), jnp.float32),
        pltpu.VMEM((out_rows,), jnp.float32),
        pltpu.VMEM_SHARED((npad, d), jnp.float32),
        pltpu.VMEM_SHARED((npad,), jnp.float32),
        pltpu.SemaphoreType.DMA,
        pltpu.SemaphoreType.DMA,
        pltpu.SemaphoreType.DMA,
        pltpu.SemaphoreType.DMA,
        pltpu.SemaphoreType.DMA,
    ]

    @functools.partial(
        pl.kernel,
        out_type=jax.ShapeDtypeStruct((_NC, npad, d), jnp.float32),
        mesh=mesh,
        scratch_types=scratch,
        compiler_params=pltpu.CompilerParams(
            needs_layout_passes=False, use_tc_tiling_on_sc=False),
    )
    def edge_kernel(h_hbm, asrc_hbm, adst_hbm, src_hbm, dst_hbm,
                    m_hbm, out_hbm, asrc_v, adst_v, denom_v, srcb_v,
                    dstb_v, p_blk, rows2_v, w_v, m_v, z_v, out_sh, den_sh,
                    sem_a, sem_g0, sem_g1, sem_s0, sem_s1):
        c = lax.axis_index("c")
        s = lax.axis_index("s")

        pltpu.sync_copy(asrc_hbm, asrc_v)
        pltpu.sync_copy(adst_hbm, adst_v)
        pltpu.sync_copy(m_hbm, m_v)

        def _zz(i, carry):
            z_v[pl.ds(i * _L, _L)] = jnp.zeros((_L,), jnp.float32)
            return carry
        lax.fori_loop(0, out_rows // _L, _zz, 0)
        pltpu.sync_copy(z_v, den_sh.at[pl.ds(s * out_rows, out_rows)])

        def _zr(r, carry):
            def _zc(g, carry2):
                rows2_v[0, r, pl.ds(g * _L, _L)] = (
                    jnp.zeros((_L,), jnp.float32))
                return carry2
            return lax.fori_loop(0, d // _L, _zc, carry)
        lax.fori_loop(0, _CB, _zr, 0)
        for q in range(out_rows // _CB):
            pltpu.sync_copy(
                rows2_v.at[0], out_sh.at[pl.ds(s * out_rows + q * _CB, _CB)])

        plsc.subcore_barrier()

        mvec = m_v[...]

        def _edge_p(rr, g, eid0):
            srcv = srcb_v[rr, pl.ds(g * _L, _L)]
            dstv = dstb_v[rr, pl.ds(g * _L, _L)]
            av = (plsc.load_gather(asrc_v, [srcv])
                  + plsc.load_gather(adst_v, [dstv]))
            av = jnp.maximum(av, 0.2 * av)
            p = jnp.exp(av - mvec)
            eid = eid0 + lax.iota(jnp.int32, _L)
            return jnp.where(eid < e_total, p, 0.0), dstv

        def _body_a(b, carry):
            row0 = b * _RB
            pltpu.sync_copy(src_hbm.at[s, pl.ds(row0, _RB)], srcb_v)
            pltpu.sync_copy(dst_hbm.at[s, pl.ds(row0, _RB)], dstb_v)
            eid_base = (s * nrows + row0) * _CH

            descs = []
            for rr in range(_RB):
                for g in range(ng):
                    p, _ = _edge_p(rr, g, eid_base + rr * _CH + g * _L)
                    p_blk[rr, pl.ds(g * _L, _L)] = p
                descs.append(pltpu.async_copy(
                    p_blk.at[rr], den_sh.at[dstb_v.at[rr]], sem_a,
                    add=True))
            for dd in descs:
                dd.wait()
            return carry
        lax.fori_loop(0, nrows // _RB, _body_a, 0)

        plsc.subcore_barrier()
        pltpu.sync_copy(den_sh, denom_v)

        g_sems = (sem_g0, sem_g1)
        s_sems = (sem_s0, sem_s1)

        def _sub_idx(rr, h2):
            return (srcb_v.at[rr, pl.ds(h2 * _CB, _CB)],
                    dstb_v.at[rr, pl.ds(h2 * _CB, _CB)])

        def _issue_gather(rr, h2, buf):
            src_ix, _ = _sub_idx(rr, h2)
            pltpu.async_copy(h_hbm.at[src_ix], rows2_v.at[buf], g_sems[buf])

        def _wait_gather(buf):
            src_ix, _ = _sub_idx(0, 0)
            pltpu.make_async_copy(
                h_hbm.at[src_ix], rows2_v.at[buf], g_sems[buf]).wait()

        def _issue_scatter(rr, h2, buf):
            _, dst_ix = _sub_idx(rr, h2)
            pltpu.async_copy(
                rows2_v.at[buf], out_sh.at[dst_ix], s_sems[buf], add=True)

        def _wait_scatter(buf):
            _, dst_ix = _sub_idx(0, 0)
            pltpu.make_async_copy(
                rows2_v.at[buf], out_sh.at[dst_ix], s_sems[buf]).wait()

        def _compute_scale(rr, h2, buf, eid_base):
            for g in range(2):
                p, dstv = _edge_p(rr, h2 * 2 + g,
                                  eid_base + rr * _CH + h2 * 2 * _L
                                  + g * _L)
                dn = plsc.load_gather(denom_v, [dstv])
                w_v[pl.ds(g * _L, _L)] = p / dn
            for g16 in range(_CB // _L):
                wg = w_v[pl.ds(g16 * _L, _L)]
                for lane in range(_L):
                    wk = wg[lane]
                    k2 = g16 * _L + lane
                    for g2 in range(d // _L):
                        sl = pl.ds(g2 * _L, _L)
                        rows2_v[buf, k2, sl] = rows2_v[buf, k2, sl] * wk

        def _body_b(b, carry):
            row0 = c * half + b * _RB
            pltpu.sync_copy(src_hbm.at[s, pl.ds(row0, _RB)], srcb_v)
            pltpu.sync_copy(dst_hbm.at[s, pl.ds(row0, _RB)], dstb_v)
            eid_base = (s * nrows + row0) * _CH

            @pl.when(b > 0)
            def _():
                _wait_scatter(0)
                _wait_scatter(1)
            _issue_gather(0, 0, 0)
            _issue_gather(0, 1, 1)

            def _chunk(rr, carry2):
                for buf in range(2):
                    _wait_gather(buf)
                    _compute_scale(rr, buf, buf, eid_base)
                    _issue_scatter(rr, buf, buf)

                @pl.when(rr < _RB - 1)
                def _():
                    for buf in range(2):
                        _wait_scatter(buf)
                        _issue_gather(rr + 1, buf, buf)
                return carry2
            lax.fori_loop(0, _RB, _chunk, 0)
            return carry
        lax.fori_loop(0, half // _RB, _body_b, 0)
        _wait_scatter(0)
        _wait_scatter(1)

        plsc.subcore_barrier()

        base = s * out_rows
        nqc = out_rows // _CB

        def _wait_store(buf):
            pltpu.make_async_copy(
                rows2_v.at[buf], out_hbm.at[c, pl.ds(base, _CB)],
                g_sems[buf]).wait()

        for q in range(nqc):
            buf = q % 2
            if q >= 2:
                _wait_store(buf)
            pltpu.sync_copy(
                out_sh.at[pl.ds(base + q * _CB, _CB)], rows2_v.at[buf])
            pltpu.async_copy(
                rows2_v.at[buf], out_hbm.at[c, pl.ds(base + q * _CB, _CB)],
                g_sems[buf])
        _wait_store((nqc - 2) % 2)
        _wait_store((nqc - 1) % 2)

    return edge_kernel


def _splat_bound(m):
    big = jnp.maximum(0.0, jnp.max(m[0]) + jnp.max(m[1]))
    return jnp.full((_L,), big, jnp.float32)


def kernel(x, edges_index, W1, att_src1, att_dst1, b1,
           W2, att_src2, att_dst2, b2):
    n, dfeat = x.shape
    dhid = W1.shape[1]
    ncls = W2.shape[1]
    d2p = ((ncls + _L - 1) // _L) * _L

    npad = ((n + _BN - 1) // _BN) * _BN
    assert npad % (_NT * _CH) == 0

    e0 = edges_index.shape[1]
    e_total = e0 + n
    grp = _NC * _NT * _CH
    epad = ((e_total + grp - 1) // grp) * grp
    echunks = epad // _CH

    pad_cnt = epad - e_total
    loop_idx = jnp.arange(n, dtype=jnp.int32)
    pad_idx = jnp.arange(pad_cnt, dtype=jnp.int32) % n
    src = jnp.concatenate([edges_index[0].astype(jnp.int32), loop_idx, pad_idx])
    dst = jnp.concatenate([edges_index[1].astype(jnp.int32), loop_idx, pad_idx])
    nrows = echunks // _NT
    src3 = src.reshape(_NT, nrows, _CH)
    dst3 = dst.reshape(_NT, nrows, _CH)

    xp = jnp.pad(x, ((0, npad - n), (0, 0)))

    h1, as1, ad1, m1 = _dense1(xp, W1,
                               att_src1.reshape(1, dhid),
                               att_dst1.reshape(1, dhid))
    part1 = _make_edge_kernel(npad, dhid, echunks, e_total)(
        h1, as1, ad1, src3, dst3, _splat_bound(m1))

    w2p = jnp.pad(W2, ((0, 0), (0, d2p - ncls)))
    as2p = jnp.pad(att_src2, (0, d2p - ncls)).reshape(1, d2p)
    ad2p = jnp.pad(att_dst2, (0, d2p - ncls)).reshape(1, d2p)
    h2, as2, ad2, m2 = _dense2(part1[0], part1[1], b1.reshape(1, dhid),
                               w2p, as2p, ad2p)
    part2 = _make_edge_kernel(npad, d2p, echunks, e_total)(
        h2, as2, ad2, src3, dst3, _splat_bound(m2))

    b2p = jnp.pad(b2, (0, d2p - ncls)).reshape(1, d2p)
    out = _combine(part2[0], part2[1], b2p)
    return out[:n, :ncls]


# speedup vs baseline: 35.8115x; 1.0108x over previous
import functools

import jax
import jax.numpy as jnp
from jax import lax
from jax.experimental import pallas as pl
from jax.experimental.pallas import tpu as pltpu
from jax.experimental.pallas import tpu_sc as plsc

_NC = 2
_NT = 16
_L = 16
_CH = 64
_CB = 32
_BN = 2048


def _dense1_body(x_ref, w_ref, as_ref, ad_ref, h_ref, a1_ref, a2_ref, m_ref):
    i = pl.program_id(0)
    h = jnp.dot(x_ref[...], w_ref[...], preferred_element_type=jnp.float32)
    h_ref[...] = h
    s = jnp.sum(h * as_ref[...], axis=1)
    t = jnp.sum(h * ad_ref[...], axis=1)
    a1_ref[...] = s
    a2_ref[...] = t

    @pl.when(i == 0)
    def _init():
        m_ref[...] = jnp.full((2, 128), -1e30, jnp.float32)

    sm = jnp.max(s.reshape(-1, 128), axis=0, keepdims=True)
    tm = jnp.max(t.reshape(-1, 128), axis=0, keepdims=True)
    m_ref[...] = jnp.maximum(m_ref[...], jnp.concatenate([sm, tm], axis=0))


def _dense1(xp, w1, as_row, ad_row):
    npad, dfeat = xp.shape
    d = w1.shape[1]
    return pl.pallas_call(
        _dense1_body,
        grid=(npad // _BN,),
        in_specs=[
            pl.BlockSpec((_BN, dfeat), lambda i: (i, 0)),
            pl.BlockSpec((dfeat, d), lambda i: (0, 0)),
            pl.BlockSpec((1, d), lambda i: (0, 0)),
            pl.BlockSpec((1, d), lambda i: (0, 0)),
        ],
        out_specs=[
            pl.BlockSpec((_BN, d), lambda i: (i, 0)),
            pl.BlockSpec((_BN,), lambda i: (i,)),
            pl.BlockSpec((_BN,), lambda i: (i,)),
            pl.BlockSpec((2, 128), lambda i: (0, 0)),
        ],
        out_shape=[
            jax.ShapeDtypeStruct((npad, d), jnp.float32),
            jax.ShapeDtypeStruct((npad,), jnp.float32),
            jax.ShapeDtypeStruct((npad,), jnp.float32),
            jax.ShapeDtypeStruct((2, 128), jnp.float32),
        ],
    )(xp, w1, as_row, ad_row)


def _dense2_body(p0_ref, p1_ref, b_ref, w_ref, as_ref, ad_ref,
                 h_ref, a1_ref, a2_ref, m_ref):
    i = pl.program_id(0)
    act = jnp.maximum(p0_ref[...] + p1_ref[...] + b_ref[...], 0.0)
    h = jnp.dot(act, w_ref[...], preferred_element_type=jnp.float32)
    h_ref[...] = h
    s = jnp.sum(h * as_ref[...], axis=1)
    t = jnp.sum(h * ad_ref[...], axis=1)
    a1_ref[...] = s
    a2_ref[...] = t

    @pl.when(i == 0)
    def _init():
        m_ref[...] = jnp.full((2, 128), -1e30, jnp.float32)

    sm = jnp.max(s.reshape(-1, 128), axis=0, keepdims=True)
    tm = jnp.max(t.reshape(-1, 128), axis=0, keepdims=True)
    m_ref[...] = jnp.maximum(m_ref[...], jnp.concatenate([sm, tm], axis=0))


def _dense2(p0, p1, b_row, w2, as_row, ad_row):
    npad, dhid = p0.shape
    d = w2.shape[1]
    return pl.pallas_call(
        _dense2_body,
        grid=(npad // _BN,),
        in_specs=[
            pl.BlockSpec((_BN, dhid), lambda i: (i, 0)),
            pl.BlockSpec((_BN, dhid), lambda i: (i, 0)),
            pl.BlockSpec((1, dhid), lambda i: (0, 0)),
            pl.BlockSpec((dhid, d), lambda i: (0, 0)),
            pl.BlockSpec((1, d), lambda i: (0, 0)),
            pl.BlockSpec((1, d), lambda i: (0, 0)),
        ],
        out_specs=[
            pl.BlockSpec((_BN, d), lambda i: (i, 0)),
            pl.BlockSpec((_BN,), lambda i: (i,)),
            pl.BlockSpec((_BN,), lambda i: (i,)),
            pl.BlockSpec((2, 128), lambda i: (0, 0)),
        ],
        out_shape=[
            jax.ShapeDtypeStruct((npad, d), jnp.float32),
            jax.ShapeDtypeStruct((npad,), jnp.float32),
            jax.ShapeDtypeStruct((npad,), jnp.float32),
            jax.ShapeDtypeStruct((2, 128), jnp.float32),
        ],
    )(p0, p1, b_row, w2, as_row, ad_row)


def _combine_body(q0_ref, q1_ref, b_ref, o_ref):
    o_ref[...] = q0_ref[...] + q1_ref[...] + b_ref[...]


def _combine(q0, q1, b_row):
    npad, d = q0.shape
    return pl.pallas_call(
        _combine_body,
        grid=(npad // _BN,),
        in_specs=[
            pl.BlockSpec((_BN, d), lambda i: (i, 0)),
            pl.BlockSpec((_BN, d), lambda i: (i, 0)),
            pl.BlockSpec((1, d), lambda i: (0, 0)),
        ],
        out_specs=pl.BlockSpec((_BN, d), lambda i: (i, 0)),
        out_shape=jax.ShapeDtypeStruct((npad, d), jnp.float32),
    )(q0, q1, b_row)


_RB = 27


@functools.cache
def _make_edge_kernel(npad, d, echunks, e_total):
    nrows = echunks // _NT
    half = nrows // 2
    out_rows = npad // _NT
    ng = _CH // _L
    assert nrows % _RB == 0 and half % _RB == 0
    mesh = plsc.VectorSubcoreMesh(
        core_axis_name="c", subcore_axis_name="s",
        num_cores=_NC, num_subcores=_NT)

    dnr = npad // _L
    scratch = [
        pltpu.VMEM((npad,), jnp.float32),
        pltpu.VMEM((npad,), jnp.float32),
        pltpu.VMEM((dnr, _L), jnp.float32),
        pltpu.VMEM((_RB, _CH), jnp.int32),
        pltpu.VMEM((_RB, _CH), jnp.int32),
        pltpu.VMEM((2, _CB, d), jnp.float32),
        pltpu.VMEM((_CB,), jnp.float32),
        pltpu.VMEM((_L,), jnp.float32),
        pltpu.VMEM((dnr // _NT, _L), jnp.float32),
        pltpu.VMEM((dnr,), jnp.int32),
        pltpu.VMEM_SHARED((npad, d), jnp.float32),
        pltpu.VMEM_SHARED((dnr, _L), jnp.float32),
        pltpu.SemaphoreType.DMA,
        pltpu.SemaphoreType.DMA,
        pltpu.SemaphoreType.DMA,
        pltpu.SemaphoreType.DMA,
    ]

    @functools.partial(
        pl.kernel,
        out_type=jax.ShapeDtypeStruct((_NC, npad, d), jnp.float32),
        mesh=mesh,
        scratch_types=scratch,
        compiler_params=pltpu.CompilerParams(
            needs_layout_passes=False, use_tc_tiling_on_sc=False),
    )
    def edge_kernel(h_hbm, asrc_hbm, adst_hbm, src_hbm, dst_hbm,
                    m_hbm, ident_hbm, out_hbm, asrc_v, adst_v, denom_v,
                    srcb_v, dstb_v, rows2_v, w_v, m_v, z_v, id_v, out_sh,
                    den_sh, sem_g0, sem_g1, sem_s0, sem_s1):
        c = lax.axis_index("c")
        s = lax.axis_index("s")
        zdr = dnr // _NT

        pltpu.sync_copy(asrc_hbm, asrc_v)
        pltpu.sync_copy(adst_hbm, adst_v)
        pltpu.sync_copy(m_hbm, m_v)

        def _zz(i, carry):
            z_v[i, pl.ds(0, _L)] = jnp.zeros((_L,), jnp.float32)
            return carry
        lax.fori_loop(0, zdr, _zz, 0)
        pltpu.sync_copy(z_v, den_sh.at[pl.ds(s * zdr, zdr)])

        def _zd(i, carry):
            denom_v[i, pl.ds(0, _L)] = jnp.zeros((_L,), jnp.float32)
            return carry
        lax.fori_loop(0, dnr, _zd, 0)

        def _zr(r, carry):
            def _zc(g, carry2):
                rows2_v[0, r, pl.ds(g * _L, _L)] = (
                    jnp.zeros((_L,), jnp.float32))
                return carry2
            return lax.fori_loop(0, d // _L, _zc, carry)
        lax.fori_loop(0, _CB, _zr, 0)
        for q in range(out_rows // _CB):
            pltpu.sync_copy(
                rows2_v.at[0], out_sh.at[pl.ds(s * out_rows + q * _CB, _CB)])

        plsc.subcore_barrier()

        mvec = m_v[...]

        def _edge_p(rr, g, eid0):
            srcv = srcb_v[rr, pl.ds(g * _L, _L)]
            dstv = dstb_v[rr, pl.ds(g * _L, _L)]
            av = (plsc.load_gather(asrc_v, [srcv])
                  + plsc.load_gather(adst_v, [dstv]))
            av = jnp.maximum(av, 0.2 * av)
            p = jnp.exp(av - mvec)
            eid = eid0 + lax.iota(jnp.int32, _L)
            return jnp.where(eid < e_total, p, 0.0), dstv

        def _body_a(b, carry):
            row0 = b * _RB
            pltpu.sync_copy(src_hbm.at[s, pl.ds(row0, _RB)], srcb_v)
            pltpu.sync_copy(dst_hbm.at[s, pl.ds(row0, _RB)], dstb_v)
            eid_base = (s * nrows + row0) * _CH

            def _row_a(rr, carry2):
                for g in range(ng):
                    p, dstv = _edge_p(rr, g, eid_base + rr * _CH + g * _L)
                    plsc.addupdate_scatter(
                        denom_v, [dstv >> 4, dstv & 15], p)
                return carry2
            lax.fori_loop(0, _RB, _row_a, 0)
            return carry
        lax.fori_loop(0, nrows // _RB, _body_a, 0)

        pltpu.sync_copy(ident_hbm, id_v)
        pltpu.sync_copy(denom_v, den_sh.at[id_v], add=True)
        plsc.subcore_barrier()
        pltpu.sync_copy(den_sh, denom_v)

        g_sems = (sem_g0, sem_g1)
        s_sems = (sem_s0, sem_s1)

        def _sub_idx(rr, h2):
            return (srcb_v.at[rr, pl.ds(h2 * _CB, _CB)],
                    dstb_v.at[rr, pl.ds(h2 * _CB, _CB)])

        def _issue_gather(rr, h2, buf):
            src_ix, _ = _sub_idx(rr, h2)
            pltpu.async_copy(h_hbm.at[src_ix], rows2_v.at[buf], g_sems[buf])

        def _wait_gather(buf):
            src_ix, _ = _sub_idx(0, 0)
            pltpu.make_async_copy(
                h_hbm.at[src_ix], rows2_v.at[buf], g_sems[buf]).wait()

        def _issue_scatter(rr, h2, buf):
            _, dst_ix = _sub_idx(rr, h2)
            pltpu.async_copy(
                rows2_v.at[buf], out_sh.at[dst_ix], s_sems[buf], add=True)

        def _wait_scatter(buf):
            _, dst_ix = _sub_idx(0, 0)
            pltpu.make_async_copy(
                rows2_v.at[buf], out_sh.at[dst_ix], s_sems[buf]).wait()

        def _compute_scale(rr, h2, buf, eid_base):
            for g in range(2):
                p, dstv = _edge_p(rr, h2 * 2 + g,
                                  eid_base + rr * _CH + h2 * 2 * _L
                                  + g * _L)
                dn = plsc.load_gather(denom_v, [dstv >> 4, dstv & 15])
                w_v[pl.ds(g * _L, _L)] = p / dn
            for g16 in range(_CB // _L):
                wg = w_v[pl.ds(g16 * _L, _L)]
                for lane in range(_L):
                    wk = wg[lane]
                    k2 = g16 * _L + lane
                    for g2 in range(d // _L):
                        sl = pl.ds(g2 * _L, _L)
                        rows2_v[buf, k2, sl] = rows2_v[buf, k2, sl] * wk

        def _body_b(b, carry):
            row0 = c * half + b * _RB
            pltpu.sync_copy(src_hbm.at[s, pl.ds(row0, _RB)], srcb_v)
            pltpu.sync_copy(dst_hbm.at[s, pl.ds(row0, _RB)], dstb_v)
            eid_base = (s * nrows + row0) * _CH

            @pl.when(b > 0)
            def _():
                _wait_scatter(0)
                _wait_scatter(1)
            _issue_gather(0, 0, 0)
            _issue_gather(0, 1, 1)

            def _chunk(rr, carry2):
                for buf in range(2):
                    _wait_gather(buf)
                    _compute_scale(rr, buf, buf, eid_base)
                    _issue_scatter(rr, buf, buf)

                @pl.when(rr < _RB - 1)
                def _():
                    for buf in range(2):
                        _wait_scatter(buf)
                        _issue_gather(rr + 1, buf, buf)
                return carry2
            lax.fori_loop(0, _RB, _chunk, 0)
            return carry
        lax.fori_loop(0, half // _RB, _body_b, 0)
        _wait_scatter(0)
        _wait_scatter(1)

        plsc.subcore_barrier()

        base = s * out_rows
        nqc = out_rows // _CB

        def _wait_store(buf):
            pltpu.make_async_copy(
                rows2_v.at[buf], out_hbm.at[c, pl.ds(base, _CB)],
                g_sems[buf]).wait()

        for q in range(nqc):
            buf = q % 2
            if q >= 2:
                _wait_store(buf)
            pltpu.sync_copy(
                out_sh.at[pl.ds(base + q * _CB, _CB)], rows2_v.at[buf])
            pltpu.async_copy(
                rows2_v.at[buf], out_hbm.at[c, pl.ds(base + q * _CB, _CB)],
                g_sems[buf])
        _wait_store((nqc - 2) % 2)
        _wait_store((nqc - 1) % 2)

    return edge_kernel


def _splat_bound(m):
    big = jnp.maximum(0.0, jnp.max(m[0]) + jnp.max(m[1]))
    return jnp.full((_L,), big, jnp.float32)


def kernel(x, edges_index, W1, att_src1, att_dst1, b1,
           W2, att_src2, att_dst2, b2):
    n, dfeat = x.shape
    dhid = W1.shape[1]
    ncls = W2.shape[1]
    d2p = ((ncls + _L - 1) // _L) * _L

    npad = ((n + _BN - 1) // _BN) * _BN
    assert npad % (_NT * _CH) == 0

    e0 = edges_index.shape[1]
    e_total = e0 + n
    grp = _NC * _NT * _CH
    epad = ((e_total + grp - 1) // grp) * grp
    echunks = epad // _CH

    pad_cnt = epad - e_total
    loop_idx = jnp.arange(n, dtype=jnp.int32)
    pad_idx = jnp.arange(pad_cnt, dtype=jnp.int32) % n
    src = jnp.concatenate([edges_index[0].astype(jnp.int32), loop_idx, pad_idx])
    dst = jnp.concatenate([edges_index[1].astype(jnp.int32), loop_idx, pad_idx])
    nrows = echunks // _NT
    src3 = src.reshape(_NT, nrows, _CH)
    dst3 = dst.reshape(_NT, nrows, _CH)

    xp = jnp.pad(x, ((0, npad - n), (0, 0)))
    ident = jnp.arange(npad // _L, dtype=jnp.int32)

    h1, as1, ad1, m1 = _dense1(xp, W1,
                               att_src1.reshape(1, dhid),
                               att_dst1.reshape(1, dhid))
    part1 = _make_edge_kernel(npad, dhid, echunks, e_total)(
        h1, as1, ad1, src3, dst3, _splat_bound(m1), ident)

    w2p = jnp.pad(W2, ((0, 0), (0, d2p - ncls)))
    as2p = jnp.pad(att_src2, (0, d2p - ncls)).reshape(1, d2p)
    ad2p = jnp.pad(att_dst2, (0, d2p - ncls)).reshape(1, d2p)
    h2, as2, ad2, m2 = _dense2(part1[0], part1[1], b1.reshape(1, dhid),
                               w2p, as2p, ad2p)
    part2 = _make_edge_kernel(npad, d2p, echunks, e_total)(
        h2, as2, ad2, src3, dst3, _splat_bound(m2), ident)

    b2p = jnp.pad(b2, (0, d2p - ncls)).reshape(1, d2p)
    out = _combine(part2[0], part2[1], b2p)
    return out[:n, :ncls]


# speedup vs baseline: 39.4907x; 1.1027x over previous
import functools

import jax
import jax.numpy as jnp
from jax import lax
from jax.experimental import pallas as pl
from jax.experimental.pallas import tpu as pltpu
from jax.experimental.pallas import tpu_sc as plsc

_NC = 2
_NT = 16
_L = 16
_CB = 32
_RB = 54
_BN = 2048


def _dense1_body(x_ref, w_ref, as_ref, ad_ref, h_ref, a1_ref, a2_ref, m_ref):
    i = pl.program_id(0)
    h = jnp.dot(x_ref[...], w_ref[...], preferred_element_type=jnp.float32)
    h_ref[...] = h
    s = jnp.sum(h * as_ref[...], axis=1)
    t = jnp.sum(h * ad_ref[...], axis=1)
    a1_ref[...] = s
    a2_ref[...] = t

    @pl.when(i == 0)
    def _init():
        m_ref[...] = jnp.full((2, 128), -1e30, jnp.float32)

    sm = jnp.max(s.reshape(-1, 128), axis=0, keepdims=True)
    tm = jnp.max(t.reshape(-1, 128), axis=0, keepdims=True)
    m_ref[...] = jnp.maximum(m_ref[...], jnp.concatenate([sm, tm], axis=0))


def _dense1(xp, w1, as_row, ad_row):
    npad, dfeat = xp.shape
    d = w1.shape[1]
    return pl.pallas_call(
        _dense1_body,
        grid=(npad // _BN,),
        in_specs=[
            pl.BlockSpec((_BN, dfeat), lambda i: (i, 0)),
            pl.BlockSpec((dfeat, d), lambda i: (0, 0)),
            pl.BlockSpec((1, d), lambda i: (0, 0)),
            pl.BlockSpec((1, d), lambda i: (0, 0)),
        ],
        out_specs=[
            pl.BlockSpec((_BN, d), lambda i: (i, 0)),
            pl.BlockSpec((_BN,), lambda i: (i,)),
            pl.BlockSpec((_BN,), lambda i: (i,)),
            pl.BlockSpec((2, 128), lambda i: (0, 0)),
        ],
        out_shape=[
            jax.ShapeDtypeStruct((npad, d), jnp.float32),
            jax.ShapeDtypeStruct((npad,), jnp.float32),
            jax.ShapeDtypeStruct((npad,), jnp.float32),
            jax.ShapeDtypeStruct((2, 128), jnp.float32),
        ],
    )(xp, w1, as_row, ad_row)


def _dense2_body(p0_ref, p1_ref, b_ref, w_ref, as_ref, ad_ref,
                 h_ref, a1_ref, a2_ref, m_ref):
    i = pl.program_id(0)
    act = jnp.maximum(p0_ref[...] + p1_ref[...] + b_ref[...], 0.0)
    h = jnp.dot(act, w_ref[...], preferred_element_type=jnp.float32)
    h_ref[...] = h
    s = jnp.sum(h * as_ref[...], axis=1)
    t = jnp.sum(h * ad_ref[...], axis=1)
    a1_ref[...] = s
    a2_ref[...] = t

    @pl.when(i == 0)
    def _init():
        m_ref[...] = jnp.full((2, 128), -1e30, jnp.float32)

    sm = jnp.max(s.reshape(-1, 128), axis=0, keepdims=True)
    tm = jnp.max(t.reshape(-1, 128), axis=0, keepdims=True)
    m_ref[...] = jnp.maximum(m_ref[...], jnp.concatenate([sm, tm], axis=0))


def _dense2(p0, p1, b_row, w2, as_row, ad_row):
    npad, dhid = p0.shape
    d = w2.shape[1]
    return pl.pallas_call(
        _dense2_body,
        grid=(npad // _BN,),
        in_specs=[
            pl.BlockSpec((_BN, dhid), lambda i: (i, 0)),
            pl.BlockSpec((_BN, dhid), lambda i: (i, 0)),
            pl.BlockSpec((1, dhid), lambda i: (0, 0)),
            pl.BlockSpec((dhid, d), lambda i: (0, 0)),
            pl.BlockSpec((1, d), lambda i: (0, 0)),
            pl.BlockSpec((1, d), lambda i: (0, 0)),
        ],
        out_specs=[
            pl.BlockSpec((_BN, d), lambda i: (i, 0)),
            pl.BlockSpec((_BN,), lambda i: (i,)),
            pl.BlockSpec((_BN,), lambda i: (i,)),
            pl.BlockSpec((2, 128), lambda i: (0, 0)),
        ],
        out_shape=[
            jax.ShapeDtypeStruct((npad, d), jnp.float32),
            jax.ShapeDtypeStruct((npad,), jnp.float32),
            jax.ShapeDtypeStruct((npad,), jnp.float32),
            jax.ShapeDtypeStruct((2, 128), jnp.float32),
        ],
    )(p0, p1, b_row, w2, as_row, ad_row)


def _combine_body(q0_ref, q1_ref, b_ref, o_ref):
    o_ref[...] = q0_ref[...] + q1_ref[...] + b_ref[...]


def _combine(q0, q1, b_row):
    npad, d = q0.shape
    return pl.pallas_call(
        _combine_body,
        grid=(npad // _BN,),
        in_specs=[
            pl.BlockSpec((_BN, d), lambda i: (i, 0)),
            pl.BlockSpec((_BN, d), lambda i: (i, 0)),
            pl.BlockSpec((1, d), lambda i: (0, 0)),
        ],
        out_specs=pl.BlockSpec((_BN, d), lambda i: (i, 0)),
        out_shape=jax.ShapeDtypeStruct((npad, d), jnp.float32),
    )(q0, q1, b_row)


@functools.cache
def _make_edge_kernel(npad, d, nrows, e_total):
    half = nrows // 2
    out_rows = npad // _NT
    ng = _CB // _L
    dnr = npad // _L
    assert nrows % _RB == 0 and half % _RB == 0 and _RB % 3 == 0
    mesh = plsc.VectorSubcoreMesh(
        core_axis_name="c", subcore_axis_name="s",
        num_cores=_NC, num_subcores=_NT)

    scratch = [
        pltpu.VMEM((npad,), jnp.float32),
        pltpu.VMEM((npad,), jnp.float32),
        pltpu.VMEM((dnr, _L), jnp.float32),
        pltpu.VMEM((_RB, _CB), jnp.int32),
        pltpu.VMEM((_RB, _CB), jnp.int32),
        pltpu.VMEM((3, _CB, d), jnp.float32),
        pltpu.VMEM((_CB,), jnp.float32),
        pltpu.VMEM((_L,), jnp.float32),
        pltpu.VMEM((dnr // _NT, _L), jnp.float32),
        pltpu.VMEM((dnr,), jnp.int32),
        pltpu.VMEM_SHARED((npad, d), jnp.float32),
        pltpu.VMEM_SHARED((dnr, _L), jnp.float32),
        [pltpu.SemaphoreType.DMA] * 3,
        [pltpu.SemaphoreType.DMA] * 3,
    ]

    @functools.partial(
        pl.kernel,
        out_type=jax.ShapeDtypeStruct((_NC, npad, d), jnp.float32),
        mesh=mesh,
        scratch_types=scratch,
        compiler_params=pltpu.CompilerParams(
            needs_layout_passes=False, use_tc_tiling_on_sc=False),
    )
    def edge_kernel(h_hbm, asrc_hbm, adst_hbm, src_hbm, dst_hbm,
                    m_hbm, ident_hbm, out_hbm, asrc_v, adst_v, denom_v,
                    srcb_v, dstb_v, rows3_v, w_v, m_v, z_v, id_v, out_sh,
                    den_sh, g_sems, s_sems):
        c = lax.axis_index("c")
        s = lax.axis_index("s")
        zdr = dnr // _NT

        pltpu.sync_copy(asrc_hbm, asrc_v)
        pltpu.sync_copy(adst_hbm, adst_v)
        pltpu.sync_copy(m_hbm, m_v)
        pltpu.sync_copy(ident_hbm, id_v)

        def _zz(i, carry):
            z_v[i, pl.ds(0, _L)] = jnp.zeros((_L,), jnp.float32)
            return carry
        lax.fori_loop(0, zdr, _zz, 0)
        pltpu.sync_copy(z_v, den_sh.at[pl.ds(s * zdr, zdr)])

        def _zd(i, carry):
            denom_v[i, pl.ds(0, _L)] = jnp.zeros((_L,), jnp.float32)
            return carry
        lax.fori_loop(0, dnr, _zd, 0)

        def _zr(r, carry):
            def _zc(g, carry2):
                rows3_v[0, r, pl.ds(g * _L, _L)] = (
                    jnp.zeros((_L,), jnp.float32))
                return carry2
            return lax.fori_loop(0, d // _L, _zc, carry)
        lax.fori_loop(0, _CB, _zr, 0)
        for q in range(out_rows // _CB):
            pltpu.sync_copy(
                rows3_v.at[0], out_sh.at[pl.ds(s * out_rows + q * _CB, _CB)])

        plsc.subcore_barrier()

        mvec = m_v[...]

        def _edge_p(rr, g, eid0):
            srcv = srcb_v[rr, pl.ds(g * _L, _L)]
            dstv = dstb_v[rr, pl.ds(g * _L, _L)]
            av = (plsc.load_gather(asrc_v, [srcv])
                  + plsc.load_gather(adst_v, [dstv]))
            av = jnp.maximum(av, 0.2 * av)
            p = jnp.exp(av - mvec)
            eid = eid0 + lax.iota(jnp.int32, _L)
            return jnp.where(eid < e_total, p, 0.0), dstv

        def _body_a(b, carry):
            row0 = b * _RB
            pltpu.sync_copy(src_hbm.at[s, pl.ds(row0, _RB)], srcb_v)
            pltpu.sync_copy(dst_hbm.at[s, pl.ds(row0, _RB)], dstb_v)
            eid_base = (s * nrows + row0) * _CB

            def _row_a(j, carry2):
                for u in range(2):
                    rr = j * 2 + u
                    for g in range(ng):
                        p, dstv = _edge_p(
                            rr, g, eid_base + rr * _CB + g * _L)
                        plsc.addupdate_scatter(
                            denom_v, [dstv >> 4, dstv & 15], p)
                return carry2
            lax.fori_loop(0, _RB // 2, _row_a, 0)
            return carry
        lax.fori_loop(0, nrows // _RB, _body_a, 0)

        pltpu.sync_copy(denom_v, den_sh.at[id_v], add=True)
        plsc.subcore_barrier()
        pltpu.sync_copy(den_sh, denom_v)

        def _issue_gather(rr, buf):
            pltpu.async_copy(
                h_hbm.at[srcb_v.at[rr]], rows3_v.at[buf], g_sems[buf])

        def _wait_gather(buf):
            pltpu.make_async_copy(
                h_hbm.at[srcb_v.at[0]], rows3_v.at[buf], g_sems[buf]).wait()

        def _issue_scatter(rr, buf):
            pltpu.async_copy(
                rows3_v.at[buf], out_sh.at[dstb_v.at[rr]], s_sems[buf],
                add=True)

        def _wait_scatter(buf):
            pltpu.make_async_copy(
                rows3_v.at[buf], out_sh.at[dstb_v.at[0]], s_sems[buf]).wait()

        def _compute_scale(rr, buf, eid_base):
            for g in range(ng):
                p, dstv = _edge_p(rr, g, eid_base + rr * _CB + g * _L)
                dn = plsc.load_gather(denom_v, [dstv >> 4, dstv & 15])
                w_v[pl.ds(g * _L, _L)] = p / dn
            for g16 in range(ng):
                wg = w_v[pl.ds(g16 * _L, _L)]
                for lane in range(_L):
                    wk = wg[lane]
                    k2 = g16 * _L + lane
                    for g2 in range(d // _L):
                        sl = pl.ds(g2 * _L, _L)
                        rows3_v[buf, k2, sl] = rows3_v[buf, k2, sl] * wk

        def _body_b(b, carry):
            row0 = c * half + b * _RB
            pltpu.sync_copy(src_hbm.at[s, pl.ds(row0, _RB)], srcb_v)
            pltpu.sync_copy(dst_hbm.at[s, pl.ds(row0, _RB)], dstb_v)
            eid_base = (s * nrows + row0) * _CB

            _issue_gather(0, 0)
            _issue_gather(1, 1)

            def _step(j, carry2):
                for u in range(3):
                    k = j * 3 + u
                    buf = u
                    _wait_gather(buf)
                    _compute_scale(k, buf, eid_base)
                    _issue_scatter(k, buf)
                    nbuf = (u + 2) % 3
                    if u == 0:
                        @pl.when((j > 0) | (b > 0))
                        def _():
                            _wait_scatter(nbuf)
                        @pl.when(j * 3 + u + 2 < _RB)
                        def _():
                            _issue_gather(k + 2, nbuf)
                    else:
                        _wait_scatter(nbuf)
                        @pl.when(j * 3 + u + 2 < _RB)
                        def _():
                            _issue_gather(k + 2, nbuf)
                return carry2
            lax.fori_loop(0, _RB // 3, _step, 0)
            return carry
        lax.fori_loop(0, half // _RB, _body_b, 0)
        _wait_scatter(2)

        plsc.subcore_barrier()

        base = s * out_rows
        nqc = out_rows // _CB

        def _wait_store(buf):
            pltpu.make_async_copy(
                rows3_v.at[buf], out_hbm.at[c, pl.ds(base, _CB)],
                g_sems[buf]).wait()

        for q in range(nqc):
            buf = q % 2
            if q >= 2:
                _wait_store(buf)
            pltpu.sync_copy(
                out_sh.at[pl.ds(base + q * _CB, _CB)], rows3_v.at[buf])
            pltpu.async_copy(
                rows3_v.at[buf], out_hbm.at[c, pl.ds(base + q * _CB, _CB)],
                g_sems[buf])
        _wait_store((nqc - 2) % 2)
        _wait_store((nqc - 1) % 2)

    return edge_kernel


def _splat_bound(m):
    big = jnp.maximum(0.0, jnp.max(m[0]) + jnp.max(m[1]))
    return jnp.full((_L,), big, jnp.float32)


def kernel(x, edges_index, W1, att_src1, att_dst1, b1,
           W2, att_src2, att_dst2, b2):
    n, dfeat = x.shape
    dhid = W1.shape[1]
    ncls = W2.shape[1]
    d2p = ((ncls + _L - 1) // _L) * _L

    npad = ((n + _BN - 1) // _BN) * _BN
    assert npad % (_NT * _CB) == 0

    e0 = edges_index.shape[1]
    e_total = e0 + n
    grp = _NC * _NT * _CB * _RB
    epad = ((e_total + grp - 1) // grp) * grp
    nrows = epad // (_NT * _CB)

    pad_cnt = epad - e_total
    loop_idx = jnp.arange(n, dtype=jnp.int32)
    pad_idx = jnp.arange(pad_cnt, dtype=jnp.int32) % n
    src = jnp.concatenate([edges_index[0].astype(jnp.int32), loop_idx, pad_idx])
    dst = jnp.concatenate([edges_index[1].astype(jnp.int32), loop_idx, pad_idx])
    src3 = src.reshape(_NT, nrows, _CB)
    dst3 = dst.reshape(_NT, nrows, _CB)

    xp = jnp.pad(x, ((0, npad - n), (0, 0)))
    ident = jnp.arange(npad // _L, dtype=jnp.int32)

    h1, as1, ad1, m1 = _dense1(xp, W1,
                               att_src1.reshape(1, dhid),
                               att_dst1.reshape(1, dhid))
    part1 = _make_edge_kernel(npad, dhid, nrows, e_total)(
        h1, as1, ad1, src3, dst3, _splat_bound(m1), ident)

    w2p = jnp.pad(W2, ((0, 0), (0, d2p - ncls)))
    as2p = jnp.pad(att_src2, (0, d2p - ncls)).reshape(1, d2p)
    ad2p = jnp.pad(att_dst2, (0, d2p - ncls)).reshape(1, d2p)
    h2, as2, ad2, m2 = _dense2(part1[0], part1[1], b1.reshape(1, dhid),
                               w2p, as2p, ad2p)
    part2 = _make_edge_kernel(npad, d2p, nrows, e_total)(
        h2, as2, ad2, src3, dst3, _splat_bound(m2), ident)

    b2p = jnp.pad(b2, (0, d2p - ncls)).reshape(1, d2p)
    out = _combine(part2[0], part2[1], b2p)
    return out[:n, :ncls]


# speedup vs baseline: 42.4928x; 1.0760x over previous
import functools

import jax
import jax.numpy as jnp
from jax import lax
from jax.experimental import pallas as pl
from jax.experimental.pallas import tpu as pltpu
from jax.experimental.pallas import tpu_sc as plsc

_NC = 2
_NT = 16
_L = 16
_CB = 32
_RB = 54
_BN = 2048


def _dense1_body(x_ref, w_ref, as_ref, ad_ref, h_ref, a1_ref, a2_ref, m_ref):
    i = pl.program_id(0)
    h = jnp.dot(x_ref[...], w_ref[...], preferred_element_type=jnp.float32)
    h_ref[...] = h
    s = jnp.sum(h * as_ref[...], axis=1)
    t = jnp.sum(h * ad_ref[...], axis=1)
    a1_ref[...] = s
    a2_ref[...] = t

    @pl.when(i == 0)
    def _init():
        m_ref[...] = jnp.full((2, 128), -1e30, jnp.float32)

    sm = jnp.max(s.reshape(-1, 128), axis=0, keepdims=True)
    tm = jnp.max(t.reshape(-1, 128), axis=0, keepdims=True)
    m_ref[...] = jnp.maximum(m_ref[...], jnp.concatenate([sm, tm], axis=0))


def _dense1(xp, w1, as_row, ad_row):
    npad, dfeat = xp.shape
    d = w1.shape[1]
    return pl.pallas_call(
        _dense1_body,
        grid=(npad // _BN,),
        in_specs=[
            pl.BlockSpec((_BN, dfeat), lambda i: (i, 0)),
            pl.BlockSpec((dfeat, d), lambda i: (0, 0)),
            pl.BlockSpec((1, d), lambda i: (0, 0)),
            pl.BlockSpec((1, d), lambda i: (0, 0)),
        ],
        out_specs=[
            pl.BlockSpec((_BN, d), lambda i: (i, 0)),
            pl.BlockSpec((_BN,), lambda i: (i,)),
            pl.BlockSpec((_BN,), lambda i: (i,)),
            pl.BlockSpec((2, 128), lambda i: (0, 0)),
        ],
        out_shape=[
            jax.ShapeDtypeStruct((npad, d), jnp.float32),
            jax.ShapeDtypeStruct((npad,), jnp.float32),
            jax.ShapeDtypeStruct((npad,), jnp.float32),
            jax.ShapeDtypeStruct((2, 128), jnp.float32),
        ],
    )(xp, w1, as_row, ad_row)


def _dense2_body(p0_ref, p1_ref, b_ref, w_ref, as_ref, ad_ref,
                 h_ref, a1_ref, a2_ref, m_ref):
    i = pl.program_id(0)
    act = jnp.maximum(p0_ref[...] + p1_ref[...] + b_ref[...], 0.0)
    h = jnp.dot(act, w_ref[...], preferred_element_type=jnp.float32)
    h_ref[...] = h
    s = jnp.sum(h * as_ref[...], axis=1)
    t = jnp.sum(h * ad_ref[...], axis=1)
    a1_ref[...] = s
    a2_ref[...] = t

    @pl.when(i == 0)
    def _init():
        m_ref[...] = jnp.full((2, 128), -1e30, jnp.float32)

    sm = jnp.max(s.reshape(-1, 128), axis=0, keepdims=True)
    tm = jnp.max(t.reshape(-1, 128), axis=0, keepdims=True)
    m_ref[...] = jnp.maximum(m_ref[...], jnp.concatenate([sm, tm], axis=0))


def _dense2(p0, p1, b_row, w2, as_row, ad_row):
    npad, dhid = p0.shape
    d = w2.shape[1]
    return pl.pallas_call(
        _dense2_body,
        grid=(npad // _BN,),
        in_specs=[
            pl.BlockSpec((_BN, dhid), lambda i: (i, 0)),
            pl.BlockSpec((_BN, dhid), lambda i: (i, 0)),
            pl.BlockSpec((1, dhid), lambda i: (0, 0)),
            pl.BlockSpec((dhid, d), lambda i: (0, 0)),
            pl.BlockSpec((1, d), lambda i: (0, 0)),
            pl.BlockSpec((1, d), lambda i: (0, 0)),
        ],
        out_specs=[
            pl.BlockSpec((_BN, d), lambda i: (i, 0)),
            pl.BlockSpec((_BN,), lambda i: (i,)),
            pl.BlockSpec((_BN,), lambda i: (i,)),
            pl.BlockSpec((2, 128), lambda i: (0, 0)),
        ],
        out_shape=[
            jax.ShapeDtypeStruct((npad, d), jnp.float32),
            jax.ShapeDtypeStruct((npad,), jnp.float32),
            jax.ShapeDtypeStruct((npad,), jnp.float32),
            jax.ShapeDtypeStruct((2, 128), jnp.float32),
        ],
    )(p0, p1, b_row, w2, as_row, ad_row)


def _combine_body(q0_ref, q1_ref, b_ref, o_ref):
    o_ref[...] = q0_ref[...] + q1_ref[...] + b_ref[...]


def _combine(q0, q1, b_row):
    npad, d = q0.shape
    return pl.pallas_call(
        _combine_body,
        grid=(npad // _BN,),
        in_specs=[
            pl.BlockSpec((_BN, d), lambda i: (i, 0)),
            pl.BlockSpec((_BN, d), lambda i: (i, 0)),
            pl.BlockSpec((1, d), lambda i: (0, 0)),
        ],
        out_specs=pl.BlockSpec((_BN, d), lambda i: (i, 0)),
        out_shape=jax.ShapeDtypeStruct((npad, d), jnp.float32),
    )(q0, q1, b_row)


@functools.cache
def _make_edge_kernel(npad, d, nrows, e_total):
    half = nrows // 2
    out_rows = npad // _NT
    ng = _CB // _L
    dnr = npad // _L
    assert nrows % _RB == 0 and half % _RB == 0 and _RB % 3 == 0
    mesh = plsc.VectorSubcoreMesh(
        core_axis_name="c", subcore_axis_name="s",
        num_cores=_NC, num_subcores=_NT)

    scratch = [
        pltpu.VMEM((npad,), jnp.float32),
        pltpu.VMEM((npad,), jnp.float32),
        pltpu.VMEM((dnr, _L), jnp.float32),
        pltpu.VMEM((_RB, _CB), jnp.int32),
        pltpu.VMEM((_RB, _CB), jnp.int32),
        pltpu.VMEM((3, _CB, d), jnp.float32),
        pltpu.VMEM((_CB,), jnp.float32),
        pltpu.VMEM((_L,), jnp.float32),
        pltpu.VMEM((dnr // _NT, _L), jnp.float32),
        pltpu.VMEM((dnr,), jnp.int32),
        pltpu.VMEM_SHARED((npad, d), jnp.float32),
        pltpu.VMEM_SHARED((dnr, _L), jnp.float32),
        [pltpu.SemaphoreType.DMA] * 3,
        [pltpu.SemaphoreType.DMA] * 3,
    ]

    @functools.partial(
        pl.kernel,
        out_type=jax.ShapeDtypeStruct((_NC, npad, d), jnp.float32),
        mesh=mesh,
        scratch_types=scratch,
        compiler_params=pltpu.CompilerParams(
            needs_layout_passes=False, use_tc_tiling_on_sc=False),
    )
    def edge_kernel(h_hbm, asrc_hbm, adst_hbm, src_hbm, dst_hbm,
                    m_hbm, ident_hbm, out_hbm, asrc_v, adst_v, denom_v,
                    srcb_v, dstb_v, rows3_v, w_v, m_v, z_v, id_v, out_sh,
                    den_sh, g_sems, s_sems):
        c = lax.axis_index("c")
        s = lax.axis_index("s")
        zdr = dnr // _NT

        pltpu.sync_copy(asrc_hbm, asrc_v)
        pltpu.sync_copy(adst_hbm, adst_v)
        pltpu.sync_copy(m_hbm, m_v)
        pltpu.sync_copy(ident_hbm, id_v)

        def _zz(i, carry):
            z_v[i, pl.ds(0, _L)] = jnp.zeros((_L,), jnp.float32)
            return carry
        lax.fori_loop(0, zdr, _zz, 0)
        pltpu.sync_copy(z_v, den_sh.at[pl.ds(s * zdr, zdr)])

        def _zd(i, carry):
            denom_v[i, pl.ds(0, _L)] = jnp.zeros((_L,), jnp.float32)
            return carry
        lax.fori_loop(0, dnr, _zd, 0)

        def _zr(r, carry):
            def _zc(g, carry2):
                rows3_v[0, r, pl.ds(g * _L, _L)] = (
                    jnp.zeros((_L,), jnp.float32))
                return carry2
            return lax.fori_loop(0, d // _L, _zc, carry)
        lax.fori_loop(0, _CB, _zr, 0)
        for q in range(out_rows // _CB):
            pltpu.sync_copy(
                rows3_v.at[0], out_sh.at[pl.ds(s * out_rows + q * _CB, _CB)])

        plsc.subcore_barrier()

        mvec = m_v[...]

        def _edge_p(rr, g, eid0):
            srcv = srcb_v[rr, pl.ds(g * _L, _L)]
            dstv = dstb_v[rr, pl.ds(g * _L, _L)]
            av = (plsc.load_gather(asrc_v, [srcv])
                  + plsc.load_gather(adst_v, [dstv]))
            av = jnp.maximum(av, 0.2 * av)
            p = jnp.exp(av - mvec)
            eid = eid0 + lax.iota(jnp.int32, _L)
            return jnp.where(eid < e_total, p, 0.0), dstv

        def _body_a(b, carry):
            row0 = b * _RB
            pltpu.sync_copy(src_hbm.at[s, pl.ds(row0, _RB)], srcb_v)
            pltpu.sync_copy(dst_hbm.at[s, pl.ds(row0, _RB)], dstb_v)
            eid_base = (s * nrows + row0) * _CB

            def _row_a(j, carry2):
                for u in range(2):
                    rr = j * 2 + u
                    for g in range(ng):
                        p, dstv = _edge_p(
                            rr, g, eid_base + rr * _CB + g * _L)
                        plsc.addupdate_scatter(
                            denom_v, [dstv >> 4, dstv & 15], p)
                return carry2
            lax.fori_loop(0, _RB // 2, _row_a, 0)
            return carry
        lax.fori_loop(0, nrows // _RB, _body_a, 0)

        pltpu.sync_copy(denom_v, den_sh.at[id_v], add=True)
        plsc.subcore_barrier()
        pltpu.sync_copy(den_sh, denom_v)

        def _issue_gather(rr, buf):
            pltpu.async_copy(
                h_hbm.at[srcb_v.at[rr]], rows3_v.at[buf], g_sems[buf])

        def _wait_gather(buf):
            pltpu.make_async_copy(
                h_hbm.at[srcb_v.at[0]], rows3_v.at[buf], g_sems[buf]).wait()

        def _issue_scatter(rr, buf):
            pltpu.async_copy(
                rows3_v.at[buf], out_sh.at[dstb_v.at[rr]], s_sems[buf],
                add=True)

        def _wait_scatter(buf):
            pltpu.make_async_copy(
                rows3_v.at[buf], out_sh.at[dstb_v.at[0]], s_sems[buf]).wait()

        def _compute_scale(rr, buf, eid_base):
            for g in range(ng):
                p, dstv = _edge_p(rr, g, eid_base + rr * _CB + g * _L)
                dn = plsc.load_gather(denom_v, [dstv >> 4, dstv & 15])
                w_v[pl.ds(g * _L, _L)] = p / dn
            for g16 in range(ng):
                wg = w_v[pl.ds(g16 * _L, _L)]
                for lane in range(_L):
                    wk = wg[lane]
                    k2 = g16 * _L + lane
                    for g2 in range(d // _L):
                        sl = pl.ds(g2 * _L, _L)
                        rows3_v[buf, k2, sl] = rows3_v[buf, k2, sl] * wk

        def _body_b(b, carry):
            row0 = c * half + b * _RB
            pltpu.sync_copy(src_hbm.at[s, pl.ds(row0, _RB)], srcb_v)
            pltpu.sync_copy(dst_hbm.at[s, pl.ds(row0, _RB)], dstb_v)
            eid_base = (s * nrows + row0) * _CB

            _issue_gather(0, 0)
            _issue_gather(1, 1)

            def _step(j, carry2):
                for u in range(3):
                    k = j * 3 + u
                    buf = u
                    nbuf = (u + 2) % 3
                    _wait_gather(buf)
                    if u == 0:
                        @pl.when((j > 0) | (b > 0))
                        def _():
                            _wait_scatter(nbuf)
                    else:
                        _wait_scatter(nbuf)

                    @pl.when(j * 3 + u + 2 < _RB)
                    def _():
                        _issue_gather(k + 2, nbuf)
                    _compute_scale(k, buf, eid_base)
                    _issue_scatter(k, buf)
                return carry2
            lax.fori_loop(0, _RB // 3, _step, 0)
            return carry
        lax.fori_loop(0, half // _RB, _body_b, 0)
        _wait_scatter(2)

        plsc.subcore_barrier()

        base = s * out_rows
        nqc = out_rows // _CB

        def _wait_store(buf):
            pltpu.make_async_copy(
                rows3_v.at[buf], out_hbm.at[c, pl.ds(base, _CB)],
                g_sems[buf]).wait()

        for q in range(nqc):
            buf = q % 2
            if q >= 2:
                _wait_store(buf)
            pltpu.sync_copy(
                out_sh.at[pl.ds(base + q * _CB, _CB)], rows3_v.at[buf])
            pltpu.async_copy(
                rows3_v.at[buf], out_hbm.at[c, pl.ds(base + q * _CB, _CB)],
                g_sems[buf])
        _wait_store((nqc - 2) % 2)
        _wait_store((nqc - 1) % 2)

    return edge_kernel


def _splat_bound(m):
    big = jnp.maximum(0.0, jnp.max(m[0]) + jnp.max(m[1]))
    return jnp.full((_L,), big, jnp.float32)


def kernel(x, edges_index, W1, att_src1, att_dst1, b1,
           W2, att_src2, att_dst2, b2):
    n, dfeat = x.shape
    dhid = W1.shape[1]
    ncls = W2.shape[1]
    d2p = ((ncls + _L - 1) // _L) * _L

    npad = ((n + _BN - 1) // _BN) * _BN
    assert npad % (_NT * _CB) == 0

    e0 = edges_index.shape[1]
    e_total = e0 + n
    grp = _NC * _NT * _CB * _RB
    epad = ((e_total + grp - 1) // grp) * grp
    nrows = epad // (_NT * _CB)

    pad_cnt = epad - e_total
    loop_idx = jnp.arange(n, dtype=jnp.int32)
    pad_idx = jnp.arange(pad_cnt, dtype=jnp.int32) % n
    src = jnp.concatenate([edges_index[0].astype(jnp.int32), loop_idx, pad_idx])
    dst = jnp.concatenate([edges_index[1].astype(jnp.int32), loop_idx, pad_idx])
    src3 = src.reshape(_NT, nrows, _CB)
    dst3 = dst.reshape(_NT, nrows, _CB)

    xp = jnp.pad(x, ((0, npad - n), (0, 0)))
    ident = jnp.arange(npad // _L, dtype=jnp.int32)

    h1, as1, ad1, m1 = _dense1(xp, W1,
                               att_src1.reshape(1, dhid),
                               att_dst1.reshape(1, dhid))
    part1 = _make_edge_kernel(npad, dhid, nrows, e_total)(
        h1, as1, ad1, src3, dst3, _splat_bound(m1), ident)

    w2p = jnp.pad(W2, ((0, 0), (0, d2p - ncls)))
    as2p = jnp.pad(att_src2, (0, d2p - ncls)).reshape(1, d2p)
    ad2p = jnp.pad(att_dst2, (0, d2p - ncls)).reshape(1, d2p)
    h2, as2, ad2, m2 = _dense2(part1[0], part1[1], b1.reshape(1, dhid),
                               w2p, as2p, ad2p)
    part2 = _make_edge_kernel(npad, d2p, nrows, e_total)(
        h2, as2, ad2, src3, dst3, _splat_bound(m2), ident)

    b2p = jnp.pad(b2, (0, d2p - ncls)).reshape(1, d2p)
    out = _combine(part2[0], part2[1], b2p)
    return out[:n, :ncls]


# speedup vs baseline: 45.8697x; 1.0795x over previous
import functools

import jax
import jax.numpy as jnp
from jax import lax
from jax.experimental import pallas as pl
from jax.experimental.pallas import tpu as pltpu
from jax.experimental.pallas import tpu_sc as plsc

_NC = 2
_NT = 16
_L = 16
_CB = 32
_RB = 54
_BN = 2048


def _dense1_body(x_ref, w_ref, as_ref, ad_ref, h_ref, a1_ref, a2_ref, m_ref):
    i = pl.program_id(0)
    h = jnp.dot(x_ref[...], w_ref[...], preferred_element_type=jnp.float32)
    h_ref[...] = h
    s = jnp.sum(h * as_ref[...], axis=1)
    t = jnp.sum(h * ad_ref[...], axis=1)
    a1_ref[...] = s
    a2_ref[...] = t

    @pl.when(i == 0)
    def _init():
        m_ref[...] = jnp.full((2, 128), -1e30, jnp.float32)

    sm = jnp.max(s.reshape(-1, 128), axis=0, keepdims=True)
    tm = jnp.max(t.reshape(-1, 128), axis=0, keepdims=True)
    m_ref[...] = jnp.maximum(m_ref[...], jnp.concatenate([sm, tm], axis=0))


def _dense1(xp, w1, as_row, ad_row):
    npad, dfeat = xp.shape
    d = w1.shape[1]
    return pl.pallas_call(
        _dense1_body,
        grid=(npad // _BN,),
        in_specs=[
            pl.BlockSpec((_BN, dfeat), lambda i: (i, 0)),
            pl.BlockSpec((dfeat, d), lambda i: (0, 0)),
            pl.BlockSpec((1, d), lambda i: (0, 0)),
            pl.BlockSpec((1, d), lambda i: (0, 0)),
        ],
        out_specs=[
            pl.BlockSpec((_BN, d), lambda i: (i, 0)),
            pl.BlockSpec((_BN,), lambda i: (i,)),
            pl.BlockSpec((_BN,), lambda i: (i,)),
            pl.BlockSpec((2, 128), lambda i: (0, 0)),
        ],
        out_shape=[
            jax.ShapeDtypeStruct((npad, d), jnp.float32),
            jax.ShapeDtypeStruct((npad,), jnp.float32),
            jax.ShapeDtypeStruct((npad,), jnp.float32),
            jax.ShapeDtypeStruct((2, 128), jnp.float32),
        ],
    )(xp, w1, as_row, ad_row)


def _dense2_body(p0_ref, p1_ref, b_ref, w_ref, as_ref, ad_ref,
                 h_ref, a1_ref, a2_ref, m_ref):
    i = pl.program_id(0)
    act = jnp.maximum(p0_ref[...] + p1_ref[...] + b_ref[...], 0.0)
    h = jnp.dot(act, w_ref[...], preferred_element_type=jnp.float32)
    h_ref[...] = h
    s = jnp.sum(h * as_ref[...], axis=1)
    t = jnp.sum(h * ad_ref[...], axis=1)
    a1_ref[...] = s
    a2_ref[...] = t

    @pl.when(i == 0)
    def _init():
        m_ref[...] = jnp.full((2, 128), -1e30, jnp.float32)

    sm = jnp.max(s.reshape(-1, 128), axis=0, keepdims=True)
    tm = jnp.max(t.reshape(-1, 128), axis=0, keepdims=True)
    m_ref[...] = jnp.maximum(m_ref[...], jnp.concatenate([sm, tm], axis=0))


def _dense2(p0, p1, b_row, w2, as_row, ad_row):
    npad, dhid = p0.shape
    d = w2.shape[1]
    return pl.pallas_call(
        _dense2_body,
        grid=(npad // _BN,),
        in_specs=[
            pl.BlockSpec((_BN, dhid), lambda i: (i, 0)),
            pl.BlockSpec((_BN, dhid), lambda i: (i, 0)),
            pl.BlockSpec((1, dhid), lambda i: (0, 0)),
            pl.BlockSpec((dhid, d), lambda i: (0, 0)),
            pl.BlockSpec((1, d), lambda i: (0, 0)),
            pl.BlockSpec((1, d), lambda i: (0, 0)),
        ],
        out_specs=[
            pl.BlockSpec((_BN, d), lambda i: (i, 0)),
            pl.BlockSpec((_BN,), lambda i: (i,)),
            pl.BlockSpec((_BN,), lambda i: (i,)),
            pl.BlockSpec((2, 128), lambda i: (0, 0)),
        ],
        out_shape=[
            jax.ShapeDtypeStruct((npad, d), jnp.float32),
            jax.ShapeDtypeStruct((npad,), jnp.float32),
            jax.ShapeDtypeStruct((npad,), jnp.float32),
            jax.ShapeDtypeStruct((2, 128), jnp.float32),
        ],
    )(p0, p1, b_row, w2, as_row, ad_row)


def _combine_body(q0_ref, q1_ref, b_ref, o_ref):
    o_ref[...] = q0_ref[...] + q1_ref[...] + b_ref[...]


def _combine(q0, q1, b_row):
    npad, d = q0.shape
    return pl.pallas_call(
        _combine_body,
        grid=(npad // _BN,),
        in_specs=[
            pl.BlockSpec((_BN, d), lambda i: (i, 0)),
            pl.BlockSpec((_BN, d), lambda i: (i, 0)),
            pl.BlockSpec((1, d), lambda i: (0, 0)),
        ],
        out_specs=pl.BlockSpec((_BN, d), lambda i: (i, 0)),
        out_shape=jax.ShapeDtypeStruct((npad, d), jnp.float32),
    )(q0, q1, b_row)


@functools.cache
def _make_edge_kernel(npad, d, nrows, e_total, cb, nbuf):
    half = nrows // 2
    out_rows = npad // _NT
    ng = cb // _L
    dnr = npad // _L
    assert nrows % _RB == 0 and half % _RB == 0 and _RB % nbuf == 0
    mesh = plsc.VectorSubcoreMesh(
        core_axis_name="c", subcore_axis_name="s",
        num_cores=_NC, num_subcores=_NT)

    scratch = [
        pltpu.VMEM((npad,), jnp.float32),
        pltpu.VMEM((npad,), jnp.float32),
        pltpu.VMEM((dnr, _L), jnp.float32),
        pltpu.VMEM((_RB, cb), jnp.int32),
        pltpu.VMEM((_RB, cb), jnp.int32),
        pltpu.VMEM((nbuf, cb, d), jnp.float32),
        pltpu.VMEM((cb,), jnp.float32),
        pltpu.VMEM((_L,), jnp.float32),
        pltpu.VMEM((dnr // _NT, _L), jnp.float32),
        pltpu.VMEM((dnr,), jnp.int32),
        pltpu.VMEM_SHARED((npad, d), jnp.float32),
        pltpu.VMEM_SHARED((dnr, _L), jnp.float32),
        [pltpu.SemaphoreType.DMA] * nbuf,
        [pltpu.SemaphoreType.DMA] * nbuf,
    ]

    @functools.partial(
        pl.kernel,
        out_type=jax.ShapeDtypeStruct((_NC, npad, d), jnp.float32),
        mesh=mesh,
        scratch_types=scratch,
        compiler_params=pltpu.CompilerParams(
            needs_layout_passes=False, use_tc_tiling_on_sc=False),
    )
    def edge_kernel(h_hbm, asrc_hbm, adst_hbm, src_hbm, dst_hbm,
                    m_hbm, ident_hbm, out_hbm, asrc_v, adst_v, denom_v,
                    srcb_v, dstb_v, rows3_v, w_v, m_v, z_v, id_v, out_sh,
                    den_sh, g_sems, s_sems):
        c = lax.axis_index("c")
        s = lax.axis_index("s")
        zdr = dnr // _NT

        pltpu.sync_copy(asrc_hbm, asrc_v)
        pltpu.sync_copy(adst_hbm, adst_v)
        pltpu.sync_copy(m_hbm, m_v)
        pltpu.sync_copy(ident_hbm, id_v)

        def _zz(i, carry):
            z_v[i, pl.ds(0, _L)] = jnp.zeros((_L,), jnp.float32)
            return carry
        lax.fori_loop(0, zdr, _zz, 0)
        pltpu.sync_copy(z_v, den_sh.at[pl.ds(s * zdr, zdr)])

        def _zd(i, carry):
            denom_v[i, pl.ds(0, _L)] = jnp.zeros((_L,), jnp.float32)
            return carry
        lax.fori_loop(0, dnr, _zd, 0)

        def _zr(r, carry):
            def _zc(g, carry2):
                rows3_v[0, r, pl.ds(g * _L, _L)] = (
                    jnp.zeros((_L,), jnp.float32))
                return carry2
            return lax.fori_loop(0, d // _L, _zc, carry)
        lax.fori_loop(0, cb, _zr, 0)
        for q in range(out_rows // cb):
            pltpu.sync_copy(
                rows3_v.at[0], out_sh.at[pl.ds(s * out_rows + q * cb, cb)])

        plsc.subcore_barrier()

        mvec = m_v[...]

        def _edge_p(rr, g, eid0):
            srcv = srcb_v[rr, pl.ds(g * _L, _L)]
            dstv = dstb_v[rr, pl.ds(g * _L, _L)]
            av = (plsc.load_gather(asrc_v, [srcv])
                  + plsc.load_gather(adst_v, [dstv]))
            av = jnp.maximum(av, 0.2 * av)
            p = jnp.exp(av - mvec)
            eid = eid0 + lax.iota(jnp.int32, _L)
            return jnp.where(eid < e_total, p, 0.0), dstv

        def _body_a(b, carry):
            row0 = b * _RB
            pltpu.sync_copy(src_hbm.at[s, pl.ds(row0, _RB)], srcb_v)
            pltpu.sync_copy(dst_hbm.at[s, pl.ds(row0, _RB)], dstb_v)
            eid_base = (s * nrows + row0) * cb

            def _row_a(j, carry2):
                for u in range(2):
                    rr = j * 2 + u
                    for g in range(ng):
                        p, dstv = _edge_p(
                            rr, g, eid_base + rr * cb + g * _L)
                        plsc.addupdate_scatter(
                            denom_v, [dstv >> 4, dstv & 15], p)
                return carry2
            lax.fori_loop(0, _RB // 2, _row_a, 0)
            return carry
        lax.fori_loop(0, nrows // _RB, _body_a, 0)

        pltpu.sync_copy(denom_v, den_sh.at[id_v], add=True)
        plsc.subcore_barrier()
        pltpu.sync_copy(den_sh, denom_v)

        def _issue_gather(rr, buf):
            pltpu.async_copy(
                h_hbm.at[srcb_v.at[rr]], rows3_v.at[buf], g_sems[buf])

        def _wait_gather(buf):
            pltpu.make_async_copy(
                h_hbm.at[srcb_v.at[0]], rows3_v.at[buf], g_sems[buf]).wait()

        def _issue_scatter(rr, buf):
            pltpu.async_copy(
                rows3_v.at[buf], out_sh.at[dstb_v.at[rr]], s_sems[buf],
                add=True)

        def _wait_scatter(buf):
            pltpu.make_async_copy(
                rows3_v.at[buf], out_sh.at[dstb_v.at[0]], s_sems[buf]).wait()

        def _compute_scale(rr, buf, eid_base):
            for g in range(ng):
                p, dstv = _edge_p(rr, g, eid_base + rr * cb + g * _L)
                dn = plsc.load_gather(denom_v, [dstv >> 4, dstv & 15])
                w_v[pl.ds(g * _L, _L)] = p / dn
            for g16 in range(ng):
                wg = w_v[pl.ds(g16 * _L, _L)]
                for lane in range(_L):
                    wk = wg[lane]
                    k2 = g16 * _L + lane
                    for g2 in range(d // _L):
                        sl = pl.ds(g2 * _L, _L)
                        rows3_v[buf, k2, sl] = rows3_v[buf, k2, sl] * wk

        def _body_b(b, carry):
            row0 = c * half + b * _RB
            pltpu.sync_copy(src_hbm.at[s, pl.ds(row0, _RB)], srcb_v)
            pltpu.sync_copy(dst_hbm.at[s, pl.ds(row0, _RB)], dstb_v)
            eid_base = (s * nrows + row0) * cb

            for pg in range(nbuf - 1):
                _issue_gather(pg, pg)

            def _step(j, carry2):
                for u in range(nbuf):
                    k = j * nbuf + u
                    buf = u
                    fbuf = (u + nbuf - 1) % nbuf
                    _wait_gather(buf)
                    if u == 0:
                        @pl.when((j > 0) | (b > 0))
                        def _():
                            _wait_scatter(fbuf)
                    else:
                        _wait_scatter(fbuf)

                    @pl.when(k + nbuf - 1 < _RB)
                    def _():
                        _issue_gather(k + nbuf - 1, fbuf)
                    _compute_scale(k, buf, eid_base)
                    _issue_scatter(k, buf)
                return carry2
            lax.fori_loop(0, _RB // nbuf, _step, 0)
            return carry
        lax.fori_loop(0, half // _RB, _body_b, 0)
        _wait_scatter(nbuf - 1)

        plsc.subcore_barrier()

        base = s * out_rows
        nqc = out_rows // cb

        def _wait_store(buf):
            pltpu.make_async_copy(
                rows3_v.at[buf], out_hbm.at[c, pl.ds(base, cb)],
                g_sems[buf]).wait()

        for q in range(nqc):
            buf = q % 2
            if q >= 2:
                _wait_store(buf)
            pltpu.sync_copy(
                out_sh.at[pl.ds(base + q * cb, cb)], rows3_v.at[buf])
            pltpu.async_copy(
                rows3_v.at[buf], out_hbm.at[c, pl.ds(base + q * cb, cb)],
                g_sems[buf])
        _wait_store((nqc - 2) % 2)
        _wait_store((nqc - 1) % 2)

    return edge_kernel


def _splat_bound(m):
    big = jnp.maximum(0.0, jnp.max(m[0]) + jnp.max(m[1]))
    return jnp.full((_L,), big, jnp.float32)


def kernel(x, edges_index, W1, att_src1, att_dst1, b1,
           W2, att_src2, att_dst2, b2):
    n, dfeat = x.shape
    dhid = W1.shape[1]
    ncls = W2.shape[1]
    d2p = ((ncls + _L - 1) // _L) * _L

    npad = ((n + _BN - 1) // _BN) * _BN
    assert npad % (_NT * _CB) == 0

    e0 = edges_index.shape[1]
    e_total = e0 + n
    grp = _NC * _NT * 64 * _RB
    epad = ((e_total + grp - 1) // grp) * grp
    nrows1 = epad // (_NT * _CB)
    nrows2 = epad // (_NT * 64)

    pad_cnt = epad - e_total
    loop_idx = jnp.arange(n, dtype=jnp.int32)
    pad_idx = jnp.arange(pad_cnt, dtype=jnp.int32) % n
    src = jnp.concatenate([edges_index[0].astype(jnp.int32), loop_idx, pad_idx])
    dst = jnp.concatenate([edges_index[1].astype(jnp.int32), loop_idx, pad_idx])
    src3 = src.reshape(_NT, nrows1, _CB)
    dst3 = dst.reshape(_NT, nrows1, _CB)
    src3b = src.reshape(_NT, nrows2, 64)
    dst3b = dst.reshape(_NT, nrows2, 64)

    xp = jnp.pad(x, ((0, npad - n), (0, 0)))
    ident = jnp.arange(npad // _L, dtype=jnp.int32)

    h1, as1, ad1, m1 = _dense1(xp, W1,
                               att_src1.reshape(1, dhid),
                               att_dst1.reshape(1, dhid))
    part1 = _make_edge_kernel(npad, dhid, nrows1, e_total, _CB, 3)(
        h1, as1, ad1, src3, dst3, _splat_bound(m1), ident)

    w2p = jnp.pad(W2, ((0, 0), (0, d2p - ncls)))
    as2p = jnp.pad(att_src2, (0, d2p - ncls)).reshape(1, d2p)
    ad2p = jnp.pad(att_dst2, (0, d2p - ncls)).reshape(1, d2p)
    h2, as2, ad2, m2 = _dense2(part1[0], part1[1], b1.reshape(1, dhid),
                               w2p, as2p, ad2p)
    part2 = _make_edge_kernel(npad, d2p, nrows2, e_total, 64, 6)(
        h2, as2, ad2, src3b, dst3b, _splat_bound(m2), ident)

    b2p = jnp.pad(b2, (0, d2p - ncls)).reshape(1, d2p)
    out = _combine(part2[0], part2[1], b2p)
    return out[:n, :ncls]
